# baseline (speedup 1.0000x reference)
import functools

import numpy as np
import jax
import jax.numpy as jnp
from jax import lax
from jax.experimental import pallas as pl
from jax.experimental.pallas import tpu as pltpu
from jax.experimental.pallas import tpu_sc as plsc

D = 11
DD = D + 1
NLEV = 16
NFEAT = 2
HS = 2 ** 19
NPTS = 131072

_PRIMES_U32 = np.array([1, 2654435761, 805459861, 3674653429, 2097192037,
                        1434869437, 2165219737, 2654435741, 2246822519,
                        3266489917, 668265263], dtype=np.uint64)[:D]
_PRIMES_I32 = [int(np.uint32(p).view(np.int32)) for p in np.uint32(_PRIMES_U32)]
_SF = [float(np.float32(DD / np.sqrt((i + 1.0) * (i + 2.0)))) for i in range(D)]
_SCALES = np.array([16.0 * (2048.0 / 16.0) ** (l / (NLEV - 1.0))
                    for l in range(NLEV)], dtype=np.float32)

PB = 64
SB = 16
LN = 128


def _wrap_i32(v: int) -> int:
    return int(np.uint32(v % (1 << 32)).view(np.int32))


def _enc_body(scale_ref, pos_ref, idx_ref, bary_ref):
    lvl = pl.program_id(0)
    scale = scale_ref[lvl, 0]
    pos = [pos_ref[j, 0] for j in range(D)]

    c = [(pos[j] * scale) * _SF[j] for j in range(D)]
    rev = [None] * D
    rev[D - 1] = c[D - 1]
    for j in range(D - 2, -1, -1):
        rev[j] = rev[j + 1] + c[j]
    elev = [rev[0]]
    for k in range(1, D):
        elev.append(rev[k] - float(k) * c[k - 1])
    zero = jnp.zeros_like(pos[0])
    elev.append(zero - float(D) * c[D - 1])

    v = [e / 12.0 for e in elev]
    greedy = []
    for k in range(DD):
        up = jnp.ceil(v[k]) * 12.0
        down = jnp.floor(v[k]) * 12.0
        greedy.append(jnp.where(up - elev[k] < elev[k] - down, up, down))
    ssum = greedy[0]
    for k in range(1, DD):
        ssum = ssum + greedy[k]
    cs = jnp.round(ssum / 12.0).astype(jnp.int32)

    diff = [elev[k] - greedy[k] for k in range(DD)]
    rank = []
    for i in range(DD):
        acc = cs
        for j in range(DD):
            if j == i:
                continue
            cond = (diff[j] >= diff[i]) if j < i else (diff[j] > diff[i])
            acc = acc + cond.astype(jnp.int32)
        rank.append(acc)
    for k in range(DD):
        tl = rank[k] < 0
        th = rank[k] >= DD
        greedy[k] = jnp.where(tl, greedy[k] + 12.0,
                              jnp.where(th, greedy[k] - 12.0, greedy[k]))
        rank[k] = jnp.where(tl, rank[k] + DD,
                            jnp.where(th, rank[k] - DD, rank[k]))

    t = [(elev[k] - greedy[k]) / 12.0 for k in range(DD)]

    A = []
    for m in range(DD):
        am = zero
        for k in range(DD):
            am = am + jnp.where(rank[k] == m, t[k], 0.0)
        A.append(am)
    bary = [None] * DD
    bary[0] = A[D] + (1.0 + (zero - A[0]))
    for r in range(1, DD):
        bary[r] = A[D - r] - A[DD - r]

    gi = [greedy[k].astype(jnp.int32) for k in range(DD)]
    gp = [gi[k] * np.int32(_PRIMES_I32[k]) for k in range(D)]
    lvl_off = lax.shift_left(lvl, 19)
    for r in range(DD):
        h = None
        for k in range(D):
            c_hi = np.int32(_wrap_i32(r * _PRIMES_I32[k]))
            c_lo = np.int32(_wrap_i32((r - DD) * _PRIMES_I32[k]))
            term = gp[k] + jnp.where(rank[k] > (D - r), c_lo, c_hi)
            h = term if h is None else h ^ term
        fidx = (h & np.int32(HS - 1)) + lvl_off
        idx_ref[0, 0, r] = fidx
        bary_ref[0, 0, r] = bary[r]


def _encode(posR, scales):
    grid = (NLEV, PB)
    out_shape = [
        jax.ShapeDtypeStruct((NLEV, PB, DD, SB, LN), jnp.int32),
        jax.ShapeDtypeStruct((NLEV, PB, DD, SB, LN), jnp.float32),
    ]
    return pl.pallas_call(
        _enc_body,
        grid=grid,
        in_specs=[
            pl.BlockSpec((NLEV, 1), lambda l, b: (0, 0),
                         memory_space=pltpu.SMEM),
            pl.BlockSpec((D, 1, SB, LN), lambda l, b: (0, b, 0, 0)),
        ],
        out_specs=[
            pl.BlockSpec((1, 1, DD, SB, LN), lambda l, b: (l, b, 0, 0, 0)),
            pl.BlockSpec((1, 1, DD, SB, LN), lambda l, b: (l, b, 0, 0, 0)),
        ],
        out_shape=out_shape,
    )(scales, posR)


NBLK = 512
BPW = NBLK // 32
BP = 256
RW = 16


def _sc_gather_body(table_ref, idx_ref, bary_ref, out_ref,
                    idx_v, bary_v, rows_v, feats_v, sem):
    nc = 2
    wid = lax.axis_index("s") * nc + lax.axis_index("c")
    lane = lax.iota(jnp.int32, 16)
    base_pat = lane * 32
    dup0 = jnp.zeros((16,), jnp.int32)

    def body(it, _):
        bi = it // NLEV
        l = it - bi * NLEV
        blk = wid * BPW + bi
        pb = blk // 8
        s2 = (blk - pb * 8) * 2
        pltpu.sync_copy(idx_ref.at[l, pb, :, pl.ds(s2, 2), :], idx_v)
        pltpu.sync_copy(bary_ref.at[l, pb, :, pl.ds(s2, 2), :], bary_v)
        handles = [
            pltpu.async_copy(table_ref.at[idx_v.at[r, a]], rows_v.at[r, a],
                             sem)
            for r in range(DD) for a in range(2)
        ]
        for h in handles:
            h.wait()
        col = 2 * l
        for g in range(BP // 16):
            a = g // 8
            boff = 16 * (g - a * 8)
            bvec = boff + lane
            widx = base_pat + (g * 512 + col)
            acc0 = jnp.zeros((16,), jnp.float32)
            acc1 = jnp.zeros((16,), jnp.float32)
            for r in range(DD):
                rvec = dup0 + r
                avec = dup0 + a
                w = bary_v[r, a, pl.ds(boff, 16)]
                f0 = plsc.load_gather(rows_v, [rvec, avec, bvec, dup0])
                f1 = plsc.load_gather(rows_v, [rvec, avec, bvec, dup0 + 1])
                acc0 = acc0 + w * f0
                acc1 = acc1 + w * f1
            plsc.store_scatter(feats_v, [widx], acc0)
            plsc.store_scatter(feats_v, [widx + 1], acc1)

        @pl.when(l == NLEV - 1)
        def _():
            pltpu.sync_copy(feats_v, out_ref.at[blk])
        return 0

    lax.fori_loop(0, BPW * NLEV, body, 0)


def _sc_gather(table, idx, bary):
    mesh = plsc.VectorSubcoreMesh(core_axis_name="c", subcore_axis_name="s")
    f = pl.kernel(
        _sc_gather_body,
        out_type=jax.ShapeDtypeStruct((NBLK, BP * 32), jnp.float32),
        mesh=mesh,
        compiler_params=pltpu.CompilerParams(needs_layout_passes=False,
                                             use_tc_tiling_on_sc=False),
        scratch_types=[
            pltpu.VMEM((DD, 2, LN), jnp.int32),
            pltpu.VMEM((DD, 2, LN), jnp.float32),
            pltpu.VMEM((DD, 2, LN, RW), jnp.float32),
            pltpu.VMEM((BP * 32,), jnp.float32),
            pltpu.SemaphoreType.DMA,
        ],
    )
    return f(table, idx, bary)


def _mlp_body(h_ref, w1_ref, b1_ref, w2_ref, b2_ref, w3_ref, b3_ref, o_ref):
    h = h_ref[...]
    h1 = jnp.maximum(jnp.dot(h, w1_ref[...],
                             preferred_element_type=jnp.float32)
                     + b1_ref[...], 0.0)
    h2 = jnp.maximum(jnp.dot(h1, w2_ref[...],
                             preferred_element_type=jnp.float32)
                     + b2_ref[...], 0.0)
    o_ref[...] = jnp.dot(h2, w3_ref[...],
                         preferred_element_type=jnp.float32) + b3_ref[...]


def _mlp(h, W1, b1, W2, b2, W3, b3):
    M = 8192
    grid = (NPTS // M,)
    full = lambda shape: pl.BlockSpec(shape, lambda i: tuple(0 for _ in shape))
    return pl.pallas_call(
        _mlp_body,
        grid=grid,
        in_specs=[
            pl.BlockSpec((M, 32), lambda i: (i, 0)),
            full((32, 64)), full((1, 64)),
            full((64, 64)), full((1, 64)),
            full((64, 1)), full((1, 1)),
        ],
        out_specs=pl.BlockSpec((M, 1), lambda i: (i, 0)),
        out_shape=jax.ShapeDtypeStruct((NPTS, 1), jnp.float32),
    )(h, W1, b1.reshape(1, 64), W2, b2.reshape(1, 64), W3, b3.reshape(1, 1))


def kernel(x, z, tables, W1, b1, W2, b2, W3, b3):
    pos = jnp.concatenate([x / 2.0 + 0.5, z], axis=-1)
    posR = pos.T.reshape(D, PB, SB, LN)
    scales = jnp.asarray(_SCALES).reshape(NLEV, 1)
    idx, bary = _encode(posR, scales)
    tab = tables.reshape(NLEV * HS, NFEAT)
    tabp = jnp.pad(tab, ((0, 0), (0, RW - NFEAT)))
    feats = _sc_gather(tabp, idx, bary)
    h = feats.reshape(NPTS, NLEV * NFEAT)
    return _mlp(h, W1, b1, W2, b2, W3, b3).squeeze(-1)


# speedup vs baseline: 1.1761x; 1.1761x over previous
import functools

import numpy as np
import jax
import jax.numpy as jnp
from jax import lax
from jax.experimental import pallas as pl
from jax.experimental.pallas import tpu as pltpu
from jax.experimental.pallas import tpu_sc as plsc

D = 11
DD = D + 1
NLEV = 16
NFEAT = 2
HS = 2 ** 19
NPTS = 131072

_PRIMES_U32 = np.array([1, 2654435761, 805459861, 3674653429, 2097192037,
                        1434869437, 2165219737, 2654435741, 2246822519,
                        3266489917, 668265263], dtype=np.uint64)[:D]
_PRIMES_I32 = [int(np.uint32(p).view(np.int32)) for p in np.uint32(_PRIMES_U32)]
_SF = [float(np.float32(DD / np.sqrt((i + 1.0) * (i + 2.0)))) for i in range(D)]
_SCALES = np.array([16.0 * (2048.0 / 16.0) ** (l / (NLEV - 1.0))
                    for l in range(NLEV)], dtype=np.float32)

PB = 64
SB = 16
LN = 128


def _wrap_i32(v: int) -> int:
    return int(np.uint32(v % (1 << 32)).view(np.int32))


def _enc_body(scale_ref, pos_ref, idx_ref, off_ref, bary_ref):
    lvl = pl.program_id(0)
    scale = scale_ref[lvl, 0]
    pos = [pos_ref[j, 0] for j in range(D)]

    c = [(pos[j] * scale) * _SF[j] for j in range(D)]
    rev = [None] * D
    rev[D - 1] = c[D - 1]
    for j in range(D - 2, -1, -1):
        rev[j] = rev[j + 1] + c[j]
    elev = [rev[0]]
    for k in range(1, D):
        elev.append(rev[k] - float(k) * c[k - 1])
    zero = jnp.zeros_like(pos[0])
    elev.append(zero - float(D) * c[D - 1])

    v = [e / 12.0 for e in elev]
    greedy = []
    for k in range(DD):
        up = jnp.ceil(v[k]) * 12.0
        down = jnp.floor(v[k]) * 12.0
        greedy.append(jnp.where(up - elev[k] < elev[k] - down, up, down))
    ssum = greedy[0]
    for k in range(1, DD):
        ssum = ssum + greedy[k]
    cs = jnp.round(ssum / 12.0).astype(jnp.int32)

    diff = [elev[k] - greedy[k] for k in range(DD)]
    rank = []
    for i in range(DD):
        acc = cs
        for j in range(DD):
            if j == i:
                continue
            cond = (diff[j] >= diff[i]) if j < i else (diff[j] > diff[i])
            acc = acc + cond.astype(jnp.int32)
        rank.append(acc)
    for k in range(DD):
        tl = rank[k] < 0
        th = rank[k] >= DD
        greedy[k] = jnp.where(tl, greedy[k] + 12.0,
                              jnp.where(th, greedy[k] - 12.0, greedy[k]))
        rank[k] = jnp.where(tl, rank[k] + DD,
                            jnp.where(th, rank[k] - DD, rank[k]))

    t = [(elev[k] - greedy[k]) / 12.0 for k in range(DD)]

    A = []
    for m in range(DD):
        am = zero
        for k in range(DD):
            am = am + jnp.where(rank[k] == m, t[k], 0.0)
        A.append(am)
    bary = [None] * DD
    bary[0] = A[D] + (1.0 + (zero - A[0]))
    for r in range(1, DD):
        bary[r] = A[D - r] - A[DD - r]

    gi = [greedy[k].astype(jnp.int32) for k in range(DD)]
    gp = [gi[k] * np.int32(_PRIMES_I32[k]) for k in range(D)]
    lvl_off = lax.shift_left(lvl, 19)
    for r in range(DD):
        h = None
        for k in range(D):
            c_hi = np.int32(_wrap_i32(r * _PRIMES_I32[k]))
            c_lo = np.int32(_wrap_i32((r - DD) * _PRIMES_I32[k]))
            term = gp[k] + jnp.where(rank[k] > (D - r), c_lo, c_hi)
            h = term if h is None else h ^ term
        fidx = (h & np.int32(HS - 1)) + lvl_off
        idx_ref[0, 0, r] = lax.shift_right_logical(fidx, 3)
        off_ref[0, 0, r] = lax.shift_left(fidx & np.int32(7), 1)
        bary_ref[0, 0, r] = bary[r]


def _encode(posR, scales):
    grid = (NLEV, PB)
    out_shape = [
        jax.ShapeDtypeStruct((NLEV, PB, DD, SB, LN), jnp.int32),
        jax.ShapeDtypeStruct((NLEV, PB, DD, SB, LN), jnp.int32),
        jax.ShapeDtypeStruct((NLEV, PB, DD, SB, LN), jnp.float32),
    ]
    return pl.pallas_call(
        _enc_body,
        grid=grid,
        in_specs=[
            pl.BlockSpec((NLEV, 1), lambda l, b: (0, 0),
                         memory_space=pltpu.SMEM),
            pl.BlockSpec((D, 1, SB, LN), lambda l, b: (0, b, 0, 0)),
        ],
        out_specs=[
            pl.BlockSpec((1, 1, DD, SB, LN), lambda l, b: (l, b, 0, 0, 0)),
            pl.BlockSpec((1, 1, DD, SB, LN), lambda l, b: (l, b, 0, 0, 0)),
            pl.BlockSpec((1, 1, DD, SB, LN), lambda l, b: (l, b, 0, 0, 0)),
        ],
        out_shape=out_shape,
    )(scales, posR)


NBLK = 512
BPW = NBLK // 32
BP = 256
RW = 16


def _sc_gather_body(table_ref, idx_ref, off_ref, bary_ref, out_ref,
                    idx_v, off_v, bary_v, rows_v, feats_v, sem):
    nc = 2
    wid = lax.axis_index("s") * nc + lax.axis_index("c")
    lane = lax.iota(jnp.int32, 16)
    base_pat = lane * 32
    dup0 = jnp.zeros((16,), jnp.int32)

    def body(it, _):
        bi = it // NLEV
        l = it - bi * NLEV
        blk = wid * BPW + bi
        pb = blk // 8
        s2 = (blk - pb * 8) * 2
        pltpu.sync_copy(idx_ref.at[l, pb, :, pl.ds(s2, 2), :], idx_v)
        pltpu.sync_copy(off_ref.at[l, pb, :, pl.ds(s2, 2), :], off_v)
        pltpu.sync_copy(bary_ref.at[l, pb, :, pl.ds(s2, 2), :], bary_v)
        handles = [
            pltpu.async_copy(table_ref.at[idx_v.at[r, a]], rows_v.at[r, a],
                             sem)
            for r in range(DD) for a in range(2)
        ]
        for h in handles:
            h.wait()
        col = 2 * l
        for g in range(BP // 16):
            a = g // 8
            boff = 16 * (g - a * 8)
            bvec = boff + lane
            widx = base_pat + (g * 512 + col)
            acc0 = jnp.zeros((16,), jnp.float32)
            acc1 = jnp.zeros((16,), jnp.float32)
            for r in range(DD):
                rvec = dup0 + r
                avec = dup0 + a
                w = bary_v[r, a, pl.ds(boff, 16)]
                off = off_v[r, a, pl.ds(boff, 16)]
                f0 = plsc.load_gather(rows_v, [rvec, avec, bvec, off])
                f1 = plsc.load_gather(rows_v, [rvec, avec, bvec, off + 1])
                acc0 = acc0 + w * f0
                acc1 = acc1 + w * f1
            plsc.store_scatter(feats_v, [widx], acc0)
            plsc.store_scatter(feats_v, [widx + 1], acc1)

        @pl.when(l == NLEV - 1)
        def _():
            pltpu.sync_copy(feats_v, out_ref.at[blk])
        return 0

    lax.fori_loop(0, BPW * NLEV, body, 0)


def _sc_gather(table, idx, off, bary):
    mesh = plsc.VectorSubcoreMesh(core_axis_name="c", subcore_axis_name="s")
    f = pl.kernel(
        _sc_gather_body,
        out_type=jax.ShapeDtypeStruct((NBLK, BP * 32), jnp.float32),
        mesh=mesh,
        compiler_params=pltpu.CompilerParams(needs_layout_passes=False,
                                             use_tc_tiling_on_sc=False),
        scratch_types=[
            pltpu.VMEM((DD, 2, LN), jnp.int32),
            pltpu.VMEM((DD, 2, LN), jnp.int32),
            pltpu.VMEM((DD, 2, LN), jnp.float32),
            pltpu.VMEM((DD, 2, LN, RW), jnp.float32),
            pltpu.VMEM((BP * 32,), jnp.float32),
            pltpu.SemaphoreType.DMA,
        ],
    )
    return f(table, idx, off, bary)


def _mlp_body(h_ref, w1_ref, b1_ref, w2_ref, b2_ref, w3_ref, b3_ref, o_ref):
    h = h_ref[...]
    h1 = jnp.maximum(jnp.dot(h, w1_ref[...],
                             preferred_element_type=jnp.float32)
                     + b1_ref[...], 0.0)
    h2 = jnp.maximum(jnp.dot(h1, w2_ref[...],
                             preferred_element_type=jnp.float32)
                     + b2_ref[...], 0.0)
    o_ref[...] = jnp.dot(h2, w3_ref[...],
                         preferred_element_type=jnp.float32) + b3_ref[...]


def _mlp(h, W1, b1, W2, b2, W3, b3):
    M = 8192
    grid = (NPTS // M,)
    full = lambda shape: pl.BlockSpec(shape, lambda i: tuple(0 for _ in shape))
    return pl.pallas_call(
        _mlp_body,
        grid=grid,
        in_specs=[
            pl.BlockSpec((M, 32), lambda i: (i, 0)),
            full((32, 64)), full((1, 64)),
            full((64, 64)), full((1, 64)),
            full((64, 1)), full((1, 1)),
        ],
        out_specs=pl.BlockSpec((M, 1), lambda i: (i, 0)),
        out_shape=jax.ShapeDtypeStruct((NPTS, 1), jnp.float32),
    )(h, W1, b1.reshape(1, 64), W2, b2.reshape(1, 64), W3, b3.reshape(1, 1))


def kernel(x, z, tables, W1, b1, W2, b2, W3, b3):
    pos = jnp.concatenate([x / 2.0 + 0.5, z], axis=-1)
    posR = pos.T.reshape(D, PB, SB, LN)
    scales = jnp.asarray(_SCALES).reshape(NLEV, 1)
    idx, off, bary = _encode(posR, scales)
    tabL = tables.reshape(NLEV * HS * NFEAT // RW, RW)
    feats = _sc_gather(tabL, idx, off, bary)
    h = feats.reshape(NPTS, NLEV * NFEAT)
    return _mlp(h, W1, b1, W2, b2, W3, b3).squeeze(-1)


# speedup vs baseline: 3.2546x; 2.7674x over previous
import functools

import numpy as np
import jax
import jax.numpy as jnp
from jax import lax
from jax.experimental import pallas as pl
from jax.experimental.pallas import tpu as pltpu
from jax.experimental.pallas import tpu_sc as plsc

D = 11
DD = D + 1
NLEV = 16
NFEAT = 2
HS = 2 ** 19
NPTS = 131072

_PRIMES_U32 = np.array([1, 2654435761, 805459861, 3674653429, 2097192037,
                        1434869437, 2165219737, 2654435741, 2246822519,
                        3266489917, 668265263], dtype=np.uint64)[:D]
_PRIMES_I32 = [int(np.uint32(p).view(np.int32)) for p in np.uint32(_PRIMES_U32)]
_SF = [float(np.float32(DD / np.sqrt((i + 1.0) * (i + 2.0)))) for i in range(D)]
_SCALES = np.array([16.0 * (2048.0 / 16.0) ** (l / (NLEV - 1.0))
                    for l in range(NLEV)], dtype=np.float32)

PB = 64
SB = 16
LN = 128


def _wrap_i32(v: int) -> int:
    return int(np.uint32(v % (1 << 32)).view(np.int32))


def _enc_body(scale_ref, pos_ref, idx_ref, id1_ref, off_ref, bary_ref):
    lvl = pl.program_id(0)
    scale = scale_ref[lvl, 0]
    pos = [pos_ref[j, 0] for j in range(D)]

    c = [(pos[j] * scale) * _SF[j] for j in range(D)]
    rev = [None] * D
    rev[D - 1] = c[D - 1]
    for j in range(D - 2, -1, -1):
        rev[j] = rev[j + 1] + c[j]
    elev = [rev[0]]
    for k in range(1, D):
        elev.append(rev[k] - float(k) * c[k - 1])
    zero = jnp.zeros_like(pos[0])
    elev.append(zero - float(D) * c[D - 1])

    v = [e / 12.0 for e in elev]
    greedy = []
    for k in range(DD):
        up = jnp.ceil(v[k]) * 12.0
        down = jnp.floor(v[k]) * 12.0
        greedy.append(jnp.where(up - elev[k] < elev[k] - down, up, down))
    ssum = greedy[0]
    for k in range(1, DD):
        ssum = ssum + greedy[k]
    cs = jnp.round(ssum / 12.0).astype(jnp.int32)

    diff = [elev[k] - greedy[k] for k in range(DD)]
    rank = []
    for i in range(DD):
        acc = cs
        for j in range(DD):
            if j == i:
                continue
            cond = (diff[j] >= diff[i]) if j < i else (diff[j] > diff[i])
            acc = acc + cond.astype(jnp.int32)
        rank.append(acc)
    for k in range(DD):
        tl = rank[k] < 0
        th = rank[k] >= DD
        greedy[k] = jnp.where(tl, greedy[k] + 12.0,
                              jnp.where(th, greedy[k] - 12.0, greedy[k]))
        rank[k] = jnp.where(tl, rank[k] + DD,
                            jnp.where(th, rank[k] - DD, rank[k]))

    t = [(elev[k] - greedy[k]) / 12.0 for k in range(DD)]

    A = []
    for m in range(DD):
        am = zero
        for k in range(DD):
            am = am + jnp.where(rank[k] == m, t[k], 0.0)
        A.append(am)
    bary = [None] * DD
    bary[0] = A[D] + (1.0 + (zero - A[0]))
    for r in range(1, DD):
        bary[r] = A[D - r] - A[DD - r]

    gi = [greedy[k].astype(jnp.int32) for k in range(DD)]
    gp = [gi[k] * np.int32(_PRIMES_I32[k]) for k in range(D)]
    lvl_off = lax.shift_left(lvl, 16)
    for r in range(DD):
        h = None
        for k in range(D):
            c_hi = np.int32(_wrap_i32(r * _PRIMES_I32[k]))
            c_lo = np.int32(_wrap_i32((r - DD) * _PRIMES_I32[k]))
            term = gp[k] + jnp.where(rank[k] > (D - r), c_lo, c_hi)
            h = term if h is None else h ^ term
        h19 = h & np.int32(HS - 1)
        line0 = (lvl_off
                 + lax.shift_left(lax.shift_right_logical(h19, 7), 4)
                 + (lax.shift_right_logical(h19, 4) & np.int32(7)))
        idx_ref[0, 0, r] = line0
        id1_ref[0, 0, r] = line0 + np.int32(8)
        off_ref[0, 0, r] = h19 & np.int32(15)
        bary_ref[0, 0, r] = bary[r]


def _encode(posR, scales):
    grid = (NLEV, PB)
    out_shape = [
        jax.ShapeDtypeStruct((NLEV, PB, DD, SB, LN), jnp.int32),
        jax.ShapeDtypeStruct((NLEV, PB, DD, SB, LN), jnp.int32),
        jax.ShapeDtypeStruct((NLEV, PB, DD, SB, LN), jnp.int32),
        jax.ShapeDtypeStruct((NLEV, PB, DD, SB, LN), jnp.float32),
    ]
    return pl.pallas_call(
        _enc_body,
        grid=grid,
        in_specs=[
            pl.BlockSpec((NLEV, 1), lambda l, b: (0, 0),
                         memory_space=pltpu.SMEM),
            pl.BlockSpec((D, 1, SB, LN), lambda l, b: (0, b, 0, 0)),
        ],
        out_specs=[
            pl.BlockSpec((1, 1, DD, SB, LN), lambda l, b: (l, b, 0, 0, 0)),
            pl.BlockSpec((1, 1, DD, SB, LN), lambda l, b: (l, b, 0, 0, 0)),
            pl.BlockSpec((1, 1, DD, SB, LN), lambda l, b: (l, b, 0, 0, 0)),
            pl.BlockSpec((1, 1, DD, SB, LN), lambda l, b: (l, b, 0, 0, 0)),
        ],
        out_shape=out_shape,
    )(scales, posR)


NBLK = 1024
BPW = NBLK // 32
BP = 128
RW = 16


def _sc_gather_body(table_ref, idx_ref, id1_ref, off_ref, bary_ref, out_ref,
                    idx_v, id1_v, off_v, bary_v, rows0_v, rows1_v,
                    feats_v, sem):
    nc = 2
    wid = lax.axis_index("s") * nc + lax.axis_index("c")
    lane = lax.iota(jnp.int32, 16)
    base_pat = lane * 32
    dup0 = jnp.zeros((16,), jnp.int32)

    def body(it, _):
        bi = it // NLEV
        l = it - bi * NLEV
        blk = wid * BPW + bi
        pb = blk // SB
        sb = blk - pb * SB
        pltpu.sync_copy(idx_ref.at[l, pb, :, sb], idx_v)
        pltpu.sync_copy(id1_ref.at[l, pb, :, sb], id1_v)
        pltpu.sync_copy(off_ref.at[l, pb, :, sb], off_v)
        pltpu.sync_copy(bary_ref.at[l, pb, :, sb], bary_v)
        handles = [
            pltpu.async_copy(table_ref.at[iv.at[r]], rv.at[r], sem)
            for r in range(DD) for iv, rv in ((idx_v, rows0_v),
                                              (id1_v, rows1_v))
        ]
        for h in handles:
            h.wait()
        col = 2 * l
        for g in range(BP // 16):
            boff = 16 * g
            bvec = boff + lane
            widx = base_pat + (g * 512 + col)
            acc0 = jnp.zeros((16,), jnp.float32)
            acc1 = jnp.zeros((16,), jnp.float32)
            for r in range(DD):
                rvec = dup0 + r
                w = bary_v[r, pl.ds(boff, 16)]
                off = off_v[r, pl.ds(boff, 16)]
                f0 = plsc.load_gather(rows0_v, [rvec, bvec, off])
                f1 = plsc.load_gather(rows1_v, [rvec, bvec, off])
                acc0 = acc0 + w * f0
                acc1 = acc1 + w * f1
            plsc.store_scatter(feats_v, [widx], acc0)
            plsc.store_scatter(feats_v, [widx + 1], acc1)

        @pl.when(l == NLEV - 1)
        def _():
            pltpu.sync_copy(feats_v, out_ref.at[blk])
        return 0

    lax.fori_loop(0, BPW * NLEV, body, 0)


def _sc_gather(table, idx, id1, off, bary):
    mesh = plsc.VectorSubcoreMesh(core_axis_name="c", subcore_axis_name="s")
    f = pl.kernel(
        _sc_gather_body,
        out_type=jax.ShapeDtypeStruct((NBLK, BP * 32), jnp.float32),
        mesh=mesh,
        compiler_params=pltpu.CompilerParams(needs_layout_passes=False,
                                             use_tc_tiling_on_sc=False),
        scratch_types=[
            pltpu.VMEM((DD, LN), jnp.int32),
            pltpu.VMEM((DD, LN), jnp.int32),
            pltpu.VMEM((DD, LN), jnp.int32),
            pltpu.VMEM((DD, LN), jnp.float32),
            pltpu.VMEM((DD, LN, RW), jnp.float32),
            pltpu.VMEM((DD, LN, RW), jnp.float32),
            pltpu.VMEM((BP * 32,), jnp.float32),
            pltpu.SemaphoreType.DMA,
        ],
    )
    return f(table, idx, id1, off, bary)


def _mlp_body(h_ref, w1_ref, b1_ref, w2_ref, b2_ref, w3_ref, b3_ref, o_ref):
    h = h_ref[...]
    h1 = jnp.maximum(jnp.dot(h, w1_ref[...],
                             preferred_element_type=jnp.float32)
                     + b1_ref[...], 0.0)
    h2 = jnp.maximum(jnp.dot(h1, w2_ref[...],
                             preferred_element_type=jnp.float32)
                     + b2_ref[...], 0.0)
    o_ref[...] = jnp.dot(h2, w3_ref[...],
                         preferred_element_type=jnp.float32) + b3_ref[...]


def _mlp(h, W1, b1, W2, b2, W3, b3):
    M = 8192
    grid = (NPTS // M,)
    full = lambda shape: pl.BlockSpec(shape, lambda i: tuple(0 for _ in shape))
    return pl.pallas_call(
        _mlp_body,
        grid=grid,
        in_specs=[
            pl.BlockSpec((M, 32), lambda i: (i, 0)),
            full((32, 64)), full((1, 64)),
            full((64, 64)), full((1, 64)),
            full((64, 1)), full((1, 1)),
        ],
        out_specs=pl.BlockSpec((M, 1), lambda i: (i, 0)),
        out_shape=jax.ShapeDtypeStruct((NPTS, 1), jnp.float32),
    )(h, W1, b1.reshape(1, 64), W2, b2.reshape(1, 64), W3, b3.reshape(1, 1))


def kernel(x, z, tables, W1, b1, W2, b2, W3, b3):
    pos = jnp.concatenate([x / 2.0 + 0.5, z], axis=-1)
    posR = pos.T.reshape(D, PB, SB, LN)
    scales = jnp.asarray(_SCALES).reshape(NLEV, 1)
    idx, id1, off, bary = _encode(posR, scales)
    tabX = tables.reshape(NLEV, HS // LN, LN, NFEAT).transpose(0, 1, 3, 2)
    tabL = tabX.reshape(NLEV * HS * NFEAT // RW, RW)
    feats = _sc_gather(tabL, idx, id1, off, bary)
    h = feats.reshape(NPTS, NLEV * NFEAT)
    return _mlp(h, W1, b1, W2, b2, W3, b3).squeeze(-1)


# speedup vs baseline: 6.1927x; 1.9027x over previous
import functools

import numpy as np
import jax
import jax.numpy as jnp
from jax import lax
from jax.experimental import pallas as pl
from jax.experimental.pallas import tpu as pltpu
from jax.experimental.pallas import tpu_sc as plsc

D = 11
DD = D + 1
NLEV = 16
NFEAT = 2
HS = 2 ** 19
NPTS = 131072

_PRIMES_U32 = np.array([1, 2654435761, 805459861, 3674653429, 2097192037,
                        1434869437, 2165219737, 2654435741, 2246822519,
                        3266489917, 668265263], dtype=np.uint64)[:D]
_PRIMES_I32 = [int(np.uint32(p).view(np.int32)) for p in np.uint32(_PRIMES_U32)]
_SF = [float(np.float32(DD / np.sqrt((i + 1.0) * (i + 2.0)))) for i in range(D)]
_SCALES = np.array([16.0 * (2048.0 / 16.0) ** (l / (NLEV - 1.0))
                    for l in range(NLEV)], dtype=np.float32)

PB = 64
SB = 16
LN = 128


def _wrap_i32(v: int) -> int:
    return int(np.uint32(v % (1 << 32)).view(np.int32))


def _enc_body(scale_ref, pos_ref, idx_ref, off_ref, bary_ref):
    lvl = pl.program_id(0)
    scale = scale_ref[lvl, 0]
    pos = [pos_ref[j, 0] for j in range(D)]

    c = [(pos[j] * scale) * _SF[j] for j in range(D)]
    rev = [None] * D
    rev[D - 1] = c[D - 1]
    for j in range(D - 2, -1, -1):
        rev[j] = rev[j + 1] + c[j]
    elev = [rev[0]]
    for k in range(1, D):
        elev.append(rev[k] - float(k) * c[k - 1])
    zero = jnp.zeros_like(pos[0])
    elev.append(zero - float(D) * c[D - 1])

    v = [e / 12.0 for e in elev]
    greedy = []
    for k in range(DD):
        up = jnp.ceil(v[k]) * 12.0
        down = jnp.floor(v[k]) * 12.0
        greedy.append(jnp.where(up - elev[k] < elev[k] - down, up, down))
    ssum = greedy[0]
    for k in range(1, DD):
        ssum = ssum + greedy[k]
    cs = jnp.round(ssum / 12.0).astype(jnp.int32)

    diff = [elev[k] - greedy[k] for k in range(DD)]
    rank = []
    for i in range(DD):
        acc = cs
        for j in range(DD):
            if j == i:
                continue
            cond = (diff[j] >= diff[i]) if j < i else (diff[j] > diff[i])
            acc = acc + cond.astype(jnp.int32)
        rank.append(acc)
    for k in range(DD):
        tl = rank[k] < 0
        th = rank[k] >= DD
        greedy[k] = jnp.where(tl, greedy[k] + 12.0,
                              jnp.where(th, greedy[k] - 12.0, greedy[k]))
        rank[k] = jnp.where(tl, rank[k] + DD,
                            jnp.where(th, rank[k] - DD, rank[k]))

    t = [(elev[k] - greedy[k]) / 12.0 for k in range(DD)]

    A = []
    for m in range(DD):
        am = zero
        for k in range(DD):
            am = am + jnp.where(rank[k] == m, t[k], 0.0)
        A.append(am)
    bary = [None] * DD
    bary[0] = A[D] + (1.0 + (zero - A[0]))
    for r in range(1, DD):
        bary[r] = A[D - r] - A[DD - r]

    gi = [greedy[k].astype(jnp.int32) for k in range(DD)]
    gp = [gi[k] * np.int32(_PRIMES_I32[k]) for k in range(D)]
    lvl_off = lax.shift_left(lvl, 19)
    for r in range(DD):
        h = None
        for k in range(D):
            c_hi = np.int32(_wrap_i32(r * _PRIMES_I32[k]))
            c_lo = np.int32(_wrap_i32((r - DD) * _PRIMES_I32[k]))
            term = gp[k] + jnp.where(rank[k] > (D - r), c_lo, c_hi)
            h = term if h is None else h ^ term
        fidx = (h & np.int32(HS - 1)) + lvl_off
        idx_ref[0, 0, r] = lax.shift_right_logical(fidx, 4)
        off_ref[0, 0, r] = fidx & np.int32(15)
        bary_ref[0, 0, r] = bary[r]


def _encode(posR, scales):
    grid = (NLEV, PB)
    out_shape = [
        jax.ShapeDtypeStruct((NLEV, PB, DD, SB, LN), jnp.int32),
        jax.ShapeDtypeStruct((NLEV, PB, DD, SB, LN), jnp.int32),
        jax.ShapeDtypeStruct((NLEV, PB, DD, SB, LN), jnp.float32),
    ]
    return pl.pallas_call(
        _enc_body,
        grid=grid,
        in_specs=[
            pl.BlockSpec((NLEV, 1), lambda l, b: (0, 0),
                         memory_space=pltpu.SMEM),
            pl.BlockSpec((D, 1, SB, LN), lambda l, b: (0, b, 0, 0)),
        ],
        out_specs=[
            pl.BlockSpec((1, 1, DD, SB, LN), lambda l, b: (l, b, 0, 0, 0)),
            pl.BlockSpec((1, 1, DD, SB, LN), lambda l, b: (l, b, 0, 0, 0)),
            pl.BlockSpec((1, 1, DD, SB, LN), lambda l, b: (l, b, 0, 0, 0)),
        ],
        out_shape=out_shape,
    )(scales, posR)


def _pack_body(x_ref, w_ref):
    xb = x_ref[0]
    x4 = xb.reshape(512, 2, 128)
    a = x4[:, 0, :]
    b = x4[:, 1, :]
    au = lax.bitcast_convert_type(a.astype(jnp.bfloat16),
                                  jnp.uint16).astype(jnp.uint32)
    bu = lax.bitcast_convert_type(b.astype(jnp.bfloat16),
                                  jnp.uint16).astype(jnp.uint32)
    w_ref[0] = lax.bitcast_convert_type(
        au | lax.shift_left(bu, jnp.uint32(16)), jnp.int32)


def _pack(tabX):
    return pl.pallas_call(
        _pack_body,
        grid=(NLEV, 8),
        in_specs=[pl.BlockSpec((1, 1024, LN), lambda l, b: (l, b, 0))],
        out_specs=pl.BlockSpec((1, 512, LN), lambda l, b: (l, b, 0)),
        out_shape=jax.ShapeDtypeStruct((NLEV, HS // LN, LN), jnp.int32),
    )(tabX)


NBLK = 1024
BPW = NBLK // 32
BP = 128
RW = 16


NIT = BPW * NLEV


def _sc_gather_body(table_ref, idx_ref, off_ref, bary_ref, out_ref,
                    idx_v, off_v, bary_v, rows_v, feats_v, sem):
    nc = 2
    wid = lax.axis_index("s") * nc + lax.axis_index("c")
    lane = lax.iota(jnp.int32, 16)
    base_pat = lane * 32
    dup0 = jnp.zeros((16,), jnp.int32)
    msk_hi = np.int32(np.uint32(0xFFFF0000).view(np.int32))

    def load_slabs(it, p):
        bi = it // NLEV
        l = it - bi * NLEV
        blk = wid * BPW + bi
        pb = blk // SB
        sb = blk - pb * SB
        pltpu.sync_copy(idx_ref.at[l, pb, :, sb], idx_v.at[p])
        pltpu.sync_copy(off_ref.at[l, pb, :, sb], off_v.at[p])
        pltpu.sync_copy(bary_ref.at[l, pb, :, sb], bary_v.at[p])

    def fire(p):
        for r in range(DD):
            pltpu.async_copy(table_ref.at[idx_v.at[p, r]],
                             rows_v.at[p, r], sem)

    def drain(p):
        for r in range(DD):
            pltpu.make_async_copy(table_ref.at[idx_v.at[p, r]],
                                  rows_v.at[p, r], sem).wait()

    def compute(it, p):
        bi = it // NLEV
        l = it - bi * NLEV
        blk = wid * BPW + bi
        col = 2 * l
        for g in range(BP // 16):
            boff = 16 * g
            bvec = boff + lane
            widx = base_pat + (g * 512 + col)
            acc0 = jnp.zeros((16,), jnp.float32)
            acc1 = jnp.zeros((16,), jnp.float32)
            for r in range(DD):
                rvec = dup0 + r
                w = bary_v[p, r, pl.ds(boff, 16)]
                off = off_v[p, r, pl.ds(boff, 16)]
                pair = plsc.load_gather(rows_v, [dup0 + p, rvec, bvec, off])
                f0 = plsc.bitcast(lax.shift_left(pair, 16), jnp.float32)
                f1 = plsc.bitcast(pair & msk_hi, jnp.float32)
                acc0 = acc0 + w * f0
                acc1 = acc1 + w * f1
            plsc.store_scatter(feats_v, [widx], acc0)
            plsc.store_scatter(feats_v, [widx + 1], acc1)

        @pl.when(l == NLEV - 1)
        def _():
            pltpu.sync_copy(feats_v, out_ref.at[blk])

    load_slabs(0, 0)
    fire(0)

    def body(t, _):
        a = 2 * t
        b = a + 1
        load_slabs(b, 1)
        fire(1)
        drain(0)
        compute(a, 0)

        @pl.when(b + 1 < NIT)
        def _():
            load_slabs(b + 1, 0)
            fire(0)

        drain(1)
        compute(b, 1)
        return 0

    lax.fori_loop(0, NIT // 2, body, 0)


def _sc_gather(table, idx, off, bary):
    mesh = plsc.VectorSubcoreMesh(core_axis_name="c", subcore_axis_name="s")
    f = pl.kernel(
        _sc_gather_body,
        out_type=jax.ShapeDtypeStruct((NBLK, BP * 32), jnp.float32),
        mesh=mesh,
        compiler_params=pltpu.CompilerParams(needs_layout_passes=False,
                                             use_tc_tiling_on_sc=False),
        scratch_types=[
            pltpu.VMEM((2, DD, LN), jnp.int32),
            pltpu.VMEM((2, DD, LN), jnp.int32),
            pltpu.VMEM((2, DD, LN), jnp.float32),
            pltpu.VMEM((2, DD, LN, RW), jnp.int32),
            pltpu.VMEM((BP * 32,), jnp.float32),
            pltpu.SemaphoreType.DMA,
        ],
    )
    return f(table, idx, off, bary)


def _mlp_body(h_ref, w1_ref, b1_ref, w2_ref, b2_ref, w3_ref, b3_ref, o_ref):
    h = h_ref[...]
    h1 = jnp.maximum(jnp.dot(h, w1_ref[...],
                             preferred_element_type=jnp.float32)
                     + b1_ref[...], 0.0)
    h2 = jnp.maximum(jnp.dot(h1, w2_ref[...],
                             preferred_element_type=jnp.float32)
                     + b2_ref[...], 0.0)
    o_ref[...] = jnp.dot(h2, w3_ref[...],
                         preferred_element_type=jnp.float32) + b3_ref[...]


def _mlp(h, W1, b1, W2, b2, W3, b3):
    M = 8192
    grid = (NPTS // M,)
    full = lambda shape: pl.BlockSpec(shape, lambda i: tuple(0 for _ in shape))
    return pl.pallas_call(
        _mlp_body,
        grid=grid,
        in_specs=[
            pl.BlockSpec((M, 32), lambda i: (i, 0)),
            full((32, 64)), full((1, 64)),
            full((64, 64)), full((1, 64)),
            full((64, 1)), full((1, 1)),
        ],
        out_specs=pl.BlockSpec((M, 1), lambda i: (i, 0)),
        out_shape=jax.ShapeDtypeStruct((NPTS, 1), jnp.float32),
    )(h, W1, b1.reshape(1, 64), W2, b2.reshape(1, 64), W3, b3.reshape(1, 1))


def kernel(x, z, tables, W1, b1, W2, b2, W3, b3):
    pos = jnp.concatenate([x / 2.0 + 0.5, z], axis=-1)
    posR = pos.T.reshape(D, PB, SB, LN)
    scales = jnp.asarray(_SCALES).reshape(NLEV, 1)
    idx, off, bary = _encode(posR, scales)
    tabX = (tables.reshape(NLEV, HS // LN, LN, NFEAT)
            .transpose(0, 1, 3, 2).reshape(NLEV, NFEAT * HS // LN, LN))
    packed = _pack(tabX)
    tabL = packed.reshape(NLEV * HS // RW, RW)
    feats = _sc_gather(tabL, idx, off, bary)
    h = feats.reshape(NPTS, NLEV * NFEAT)
    return _mlp(h, W1, b1, W2, b2, W3, b3).squeeze(-1)


# speedup vs baseline: 7.2791x; 1.1754x over previous
import functools

import numpy as np
import jax
import jax.numpy as jnp
from jax import lax
from jax.experimental import pallas as pl
from jax.experimental.pallas import tpu as pltpu
from jax.experimental.pallas import tpu_sc as plsc

D = 11
DD = D + 1
NLEV = 16
NFEAT = 2
HS = 2 ** 19
NPTS = 131072

_PRIMES_U32 = np.array([1, 2654435761, 805459861, 3674653429, 2097192037,
                        1434869437, 2165219737, 2654435741, 2246822519,
                        3266489917, 668265263], dtype=np.uint64)[:D]
_PRIMES_I32 = [int(np.uint32(p).view(np.int32)) for p in np.uint32(_PRIMES_U32)]
_SF = [float(np.float32(DD / np.sqrt((i + 1.0) * (i + 2.0)))) for i in range(D)]
_SCALES = np.array([16.0 * (2048.0 / 16.0) ** (l / (NLEV - 1.0))
                    for l in range(NLEV)], dtype=np.float32)

PB = 64
SB = 16
LN = 128


def _wrap_i32(v: int) -> int:
    return int(np.uint32(v % (1 << 32)).view(np.int32))


def _enc_body(scale_ref, pos_ref, idx_ref, off_ref, bary_ref):
    lvl = pl.program_id(0)
    scale = scale_ref[lvl, 0]
    pos = [pos_ref[j, 0] for j in range(D)]

    c = [(pos[j] * scale) * _SF[j] for j in range(D)]
    rev = [None] * D
    rev[D - 1] = c[D - 1]
    for j in range(D - 2, -1, -1):
        rev[j] = rev[j + 1] + c[j]
    elev = [rev[0]]
    for k in range(1, D):
        elev.append(rev[k] - float(k) * c[k - 1])
    zero = jnp.zeros_like(pos[0])
    elev.append(zero - float(D) * c[D - 1])

    v = [e / 12.0 for e in elev]
    greedy = []
    for k in range(DD):
        up = jnp.ceil(v[k]) * 12.0
        down = jnp.floor(v[k]) * 12.0
        greedy.append(jnp.where(up - elev[k] < elev[k] - down, up, down))
    ssum = greedy[0]
    for k in range(1, DD):
        ssum = ssum + greedy[k]
    cs = jnp.round(ssum / 12.0).astype(jnp.int32)

    diff = [elev[k] - greedy[k] for k in range(DD)]
    rank = []
    for i in range(DD):
        acc = cs
        for j in range(DD):
            if j == i:
                continue
            cond = (diff[j] >= diff[i]) if j < i else (diff[j] > diff[i])
            acc = acc + cond.astype(jnp.int32)
        rank.append(acc)
    for k in range(DD):
        tl = rank[k] < 0
        th = rank[k] >= DD
        greedy[k] = jnp.where(tl, greedy[k] + 12.0,
                              jnp.where(th, greedy[k] - 12.0, greedy[k]))
        rank[k] = jnp.where(tl, rank[k] + DD,
                            jnp.where(th, rank[k] - DD, rank[k]))

    t = [(elev[k] - greedy[k]) / 12.0 for k in range(DD)]

    A = []
    for m in range(DD):
        am = zero
        for k in range(DD):
            am = am + jnp.where(rank[k] == m, t[k], 0.0)
        A.append(am)
    bary = [None] * DD
    bary[0] = A[D] + (1.0 + (zero - A[0]))
    for r in range(1, DD):
        bary[r] = A[D - r] - A[DD - r]

    gi = [greedy[k].astype(jnp.int32) for k in range(DD)]
    gp = [gi[k] * np.int32(_PRIMES_I32[k]) for k in range(D)]
    lvl_off = lax.shift_left(lvl, 19)
    for r in range(DD):
        h = None
        for k in range(D):
            c_hi = np.int32(_wrap_i32(r * _PRIMES_I32[k]))
            c_lo = np.int32(_wrap_i32((r - DD) * _PRIMES_I32[k]))
            term = gp[k] + jnp.where(rank[k] > (D - r), c_lo, c_hi)
            h = term if h is None else h ^ term
        fidx = (h & np.int32(HS - 1)) + lvl_off
        idx_ref[0, 0, r] = lax.shift_right_logical(fidx, 4)
        off_ref[0, 0, r] = fidx & np.int32(15)
        bary_ref[0, 0, r] = bary[r]


def _encode(posR, scales, npb):
    grid = (NLEV, npb)
    out_shape = [
        jax.ShapeDtypeStruct((NLEV, npb, DD, SB, LN), jnp.int32),
        jax.ShapeDtypeStruct((NLEV, npb, DD, SB, LN), jnp.int32),
        jax.ShapeDtypeStruct((NLEV, npb, DD, SB, LN), jnp.float32),
    ]
    return pl.pallas_call(
        _enc_body,
        grid=grid,
        in_specs=[
            pl.BlockSpec((NLEV, 1), lambda l, b: (0, 0),
                         memory_space=pltpu.SMEM),
            pl.BlockSpec((D, 1, SB, LN), lambda l, b: (0, b, 0, 0)),
        ],
        out_specs=[
            pl.BlockSpec((1, 1, DD, SB, LN), lambda l, b: (l, b, 0, 0, 0)),
            pl.BlockSpec((1, 1, DD, SB, LN), lambda l, b: (l, b, 0, 0, 0)),
            pl.BlockSpec((1, 1, DD, SB, LN), lambda l, b: (l, b, 0, 0, 0)),
        ],
        out_shape=out_shape,
    )(scales, posR)


def _pack_body(x_ref, w_ref):
    xb = x_ref[0]
    x4 = xb.reshape(512, 2, 128)
    a = x4[:, 0, :]
    b = x4[:, 1, :]
    au = lax.bitcast_convert_type(a.astype(jnp.bfloat16),
                                  jnp.uint16).astype(jnp.uint32)
    bu = lax.bitcast_convert_type(b.astype(jnp.bfloat16),
                                  jnp.uint16).astype(jnp.uint32)
    w_ref[0] = lax.bitcast_convert_type(
        au | lax.shift_left(bu, jnp.uint32(16)), jnp.int32)


def _pack(tabX):
    return pl.pallas_call(
        _pack_body,
        grid=(NLEV, 8),
        in_specs=[pl.BlockSpec((1, 1024, LN), lambda l, b: (l, b, 0))],
        out_specs=pl.BlockSpec((1, 512, LN), lambda l, b: (l, b, 0)),
        out_shape=jax.ShapeDtypeStruct((NLEV, HS // LN, LN), jnp.int32),
    )(tabX)


BP = 128
RW = 16


def _sc_gather_body(BPW, table_ref, idx_ref, off_ref, bary_ref, out_ref,
                    idx_v, off_v, bary_v, rows_v, feats_v, sem):
    NIT = BPW * NLEV
    nc = 2
    wid = lax.axis_index("s") * nc + lax.axis_index("c")
    lane = lax.iota(jnp.int32, 16)
    base_pat = lane * 32
    dup0 = jnp.zeros((16,), jnp.int32)
    msk_hi = np.int32(np.uint32(0xFFFF0000).view(np.int32))

    def load_slabs(it, p):
        bi = it // NLEV
        l = it - bi * NLEV
        blk = wid * BPW + bi
        pb = blk // SB
        sb = blk - pb * SB
        pltpu.sync_copy(idx_ref.at[l, pb, :, sb], idx_v.at[p])
        pltpu.sync_copy(off_ref.at[l, pb, :, sb], off_v.at[p])
        pltpu.sync_copy(bary_ref.at[l, pb, :, sb], bary_v.at[p])

    def fire(p):
        for r in range(DD):
            pltpu.async_copy(table_ref.at[idx_v.at[p, r]],
                             rows_v.at[p, r], sem)

    def drain(p):
        for r in range(DD):
            pltpu.make_async_copy(table_ref.at[idx_v.at[p, r]],
                                  rows_v.at[p, r], sem).wait()

    def compute(it, p):
        bi = it // NLEV
        l = it - bi * NLEV
        blk = wid * BPW + bi
        col = 2 * l
        for g in range(BP // 16):
            boff = 16 * g
            bvec = boff + lane
            widx = base_pat + (g * 512 + col)
            acc0 = jnp.zeros((16,), jnp.float32)
            acc1 = jnp.zeros((16,), jnp.float32)
            for r in range(DD):
                rvec = dup0 + r
                w = bary_v[p, r, pl.ds(boff, 16)]
                off = off_v[p, r, pl.ds(boff, 16)]
                pair = plsc.load_gather(rows_v, [dup0 + p, rvec, bvec, off])
                f0 = plsc.bitcast(lax.shift_left(pair, 16), jnp.float32)
                f1 = plsc.bitcast(pair & msk_hi, jnp.float32)
                acc0 = acc0 + w * f0
                acc1 = acc1 + w * f1
            plsc.store_scatter(feats_v, [widx], acc0)
            plsc.store_scatter(feats_v, [widx + 1], acc1)

        @pl.when(l == NLEV - 1)
        def _():
            pltpu.sync_copy(feats_v, out_ref.at[blk])

    load_slabs(0, 0)
    fire(0)

    def body(t, _):
        a = 2 * t
        b = a + 1
        load_slabs(b, 1)
        fire(1)
        drain(0)
        compute(a, 0)

        @pl.when(b + 1 < NIT)
        def _():
            load_slabs(b + 1, 0)
            fire(0)

        drain(1)
        compute(b, 1)
        return 0

    lax.fori_loop(0, NIT // 2, body, 0)


def _sc_gather(table, idx, off, bary):
    nblk = idx.shape[1] * SB
    mesh = plsc.VectorSubcoreMesh(core_axis_name="c", subcore_axis_name="s")
    f = pl.kernel(
        functools.partial(_sc_gather_body, nblk // 32),
        out_type=jax.ShapeDtypeStruct((nblk, BP * 32), jnp.float32),
        mesh=mesh,
        compiler_params=pltpu.CompilerParams(needs_layout_passes=False,
                                             use_tc_tiling_on_sc=False),
        scratch_types=[
            pltpu.VMEM((2, DD, LN), jnp.int32),
            pltpu.VMEM((2, DD, LN), jnp.int32),
            pltpu.VMEM((2, DD, LN), jnp.float32),
            pltpu.VMEM((2, DD, LN, RW), jnp.int32),
            pltpu.VMEM((BP * 32,), jnp.float32),
            pltpu.SemaphoreType.DMA,
        ],
    )
    return f(table, idx, off, bary)


def _mlp_body(h_ref, w1_ref, b1_ref, w2_ref, b2_ref, w3_ref, b3_ref, o_ref):
    h = h_ref[...]
    h1 = jnp.maximum(jnp.dot(h, w1_ref[...],
                             preferred_element_type=jnp.float32)
                     + b1_ref[...], 0.0)
    h2 = jnp.maximum(jnp.dot(h1, w2_ref[...],
                             preferred_element_type=jnp.float32)
                     + b2_ref[...], 0.0)
    o_ref[...] = jnp.dot(h2, w3_ref[...],
                         preferred_element_type=jnp.float32) + b3_ref[...]


def _mlp(h, W1, b1, W2, b2, W3, b3):
    M = 8192
    npts = h.shape[0]
    grid = (npts // M,)
    full = lambda shape: pl.BlockSpec(shape, lambda i: tuple(0 for _ in shape))
    return pl.pallas_call(
        _mlp_body,
        grid=grid,
        in_specs=[
            pl.BlockSpec((M, 32), lambda i: (i, 0)),
            full((32, 64)), full((1, 64)),
            full((64, 64)), full((1, 64)),
            full((64, 1)), full((1, 1)),
        ],
        out_specs=pl.BlockSpec((M, 1), lambda i: (i, 0)),
        out_shape=jax.ShapeDtypeStruct((npts, 1), jnp.float32),
    )(h, W1, b1.reshape(1, 64), W2, b2.reshape(1, 64), W3, b3.reshape(1, 1))


def kernel(x, z, tables, W1, b1, W2, b2, W3, b3):
    pos = jnp.concatenate([x / 2.0 + 0.5, z], axis=-1)
    posR = pos.T.reshape(D, PB, SB, LN)
    scales = jnp.asarray(_SCALES).reshape(NLEV, 1)
    tabX = (tables.reshape(NLEV, HS // LN, LN, NFEAT)
            .transpose(0, 1, 3, 2).reshape(NLEV, NFEAT * HS // LN, LN))
    packed = _pack(tabX)
    tabL = packed.reshape(NLEV * HS // RW, RW)
    outs = []
    HPB = PB // 2
    for hi in range(2):
        posH = posR[:, hi * HPB:(hi + 1) * HPB]
        idx, off, bary = _encode(posH, scales, HPB)
        feats = _sc_gather(tabL, idx, off, bary)
        h = feats.reshape(HPB * 2048, NLEV * NFEAT)
        outs.append(_mlp(h, W1, b1, W2, b2, W3, b3))
    return jnp.concatenate(outs, axis=0).squeeze(-1)


# speedup vs baseline: 7.9250x; 1.0887x over previous
import functools

import numpy as np
import jax
import jax.numpy as jnp
from jax import lax
from jax.experimental import pallas as pl
from jax.experimental.pallas import tpu as pltpu
from jax.experimental.pallas import tpu_sc as plsc

D = 11
DD = D + 1
NLEV = 16
NFEAT = 2
HS = 2 ** 19
NPTS = 131072

_PRIMES_U32 = np.array([1, 2654435761, 805459861, 3674653429, 2097192037,
                        1434869437, 2165219737, 2654435741, 2246822519,
                        3266489917, 668265263], dtype=np.uint64)[:D]
_PRIMES_I32 = [int(np.uint32(p).view(np.int32)) for p in np.uint32(_PRIMES_U32)]
_SF = [float(np.float32(DD / np.sqrt((i + 1.0) * (i + 2.0)))) for i in range(D)]
_SCALES = np.array([16.0 * (2048.0 / 16.0) ** (l / (NLEV - 1.0))
                    for l in range(NLEV)], dtype=np.float32)

PB = 64
SB = 16
LN = 128


def _wrap_i32(v: int) -> int:
    return int(np.uint32(v % (1 << 32)).view(np.int32))


def _enc_body(scale_ref, pos_ref, idx_ref, off_ref, bary_ref):
    lvl = pl.program_id(0)
    scale = scale_ref[lvl, 0]
    pos = [pos_ref[j, 0] for j in range(D)]

    c = [(pos[j] * scale) * _SF[j] for j in range(D)]
    rev = [None] * D
    rev[D - 1] = c[D - 1]
    for j in range(D - 2, -1, -1):
        rev[j] = rev[j + 1] + c[j]
    elev = [rev[0]]
    for k in range(1, D):
        elev.append(rev[k] - float(k) * c[k - 1])
    zero = jnp.zeros_like(pos[0])
    elev.append(zero - float(D) * c[D - 1])

    v = [e / 12.0 for e in elev]
    greedy = []
    for k in range(DD):
        up = jnp.ceil(v[k]) * 12.0
        down = jnp.floor(v[k]) * 12.0
        greedy.append(jnp.where(up - elev[k] < elev[k] - down, up, down))
    ssum = greedy[0]
    for k in range(1, DD):
        ssum = ssum + greedy[k]
    cs = jnp.round(ssum / 12.0).astype(jnp.int32)

    diff = [elev[k] - greedy[k] for k in range(DD)]
    rank = []
    for i in range(DD):
        acc = cs
        for j in range(DD):
            if j == i:
                continue
            cond = (diff[j] >= diff[i]) if j < i else (diff[j] > diff[i])
            acc = acc + cond.astype(jnp.int32)
        rank.append(acc)
    for k in range(DD):
        tl = rank[k] < 0
        th = rank[k] >= DD
        greedy[k] = jnp.where(tl, greedy[k] + 12.0,
                              jnp.where(th, greedy[k] - 12.0, greedy[k]))
        rank[k] = jnp.where(tl, rank[k] + DD,
                            jnp.where(th, rank[k] - DD, rank[k]))

    t = [(elev[k] - greedy[k]) / 12.0 for k in range(DD)]

    A = []
    for m in range(DD):
        am = zero
        for k in range(DD):
            am = am + jnp.where(rank[k] == m, t[k], 0.0)
        A.append(am)
    bary = [None] * DD
    bary[0] = A[D] + (1.0 + (zero - A[0]))
    for r in range(1, DD):
        bary[r] = A[D - r] - A[DD - r]

    gi = [greedy[k].astype(jnp.int32) for k in range(DD)]
    gp = [gi[k] * np.int32(_PRIMES_I32[k]) for k in range(D)]
    lvl_off = lax.shift_left(lvl, 19)
    for r in range(DD):
        h = None
        for k in range(D):
            c_hi = np.int32(_wrap_i32(r * _PRIMES_I32[k]))
            c_lo = np.int32(_wrap_i32((r - DD) * _PRIMES_I32[k]))
            term = gp[k] + jnp.where(rank[k] > (D - r), c_lo, c_hi)
            h = term if h is None else h ^ term
        fidx = (h & np.int32(HS - 1)) + lvl_off
        idx_ref[0, 0, r] = lax.shift_right_logical(fidx, 4)
        off_ref[0, 0, r] = fidx & np.int32(15)
        bary_ref[0, 0, r] = bary[r]


def _encode(posR, scales, npb):
    grid = (NLEV, npb)
    out_shape = [
        jax.ShapeDtypeStruct((NLEV, npb, DD, SB, LN), jnp.int32),
        jax.ShapeDtypeStruct((NLEV, npb, DD, SB, LN), jnp.int32),
        jax.ShapeDtypeStruct((NLEV, npb, DD, SB, LN), jnp.float32),
    ]
    return pl.pallas_call(
        _enc_body,
        grid=grid,
        in_specs=[
            pl.BlockSpec((NLEV, 1), lambda l, b: (0, 0),
                         memory_space=pltpu.SMEM),
            pl.BlockSpec((D, 1, SB, LN), lambda l, b: (0, b, 0, 0)),
        ],
        out_specs=[
            pl.BlockSpec((1, 1, DD, SB, LN), lambda l, b: (l, b, 0, 0, 0)),
            pl.BlockSpec((1, 1, DD, SB, LN), lambda l, b: (l, b, 0, 0, 0)),
            pl.BlockSpec((1, 1, DD, SB, LN), lambda l, b: (l, b, 0, 0, 0)),
        ],
        out_shape=out_shape,
    )(scales, posR)


def _pack_body(x_ref, w_ref):
    xb = x_ref[0]
    x4 = xb.reshape(512, 2, 128)
    a = x4[:, 0, :]
    b = x4[:, 1, :]
    au = lax.bitcast_convert_type(a.astype(jnp.bfloat16),
                                  jnp.uint16).astype(jnp.uint32)
    bu = lax.bitcast_convert_type(b.astype(jnp.bfloat16),
                                  jnp.uint16).astype(jnp.uint32)
    w_ref[0] = lax.bitcast_convert_type(
        au | lax.shift_left(bu, jnp.uint32(16)), jnp.int32)


def _pack(tabX):
    return pl.pallas_call(
        _pack_body,
        grid=(NLEV, 8),
        in_specs=[pl.BlockSpec((1, 1024, LN), lambda l, b: (l, b, 0))],
        out_specs=pl.BlockSpec((1, 512, LN), lambda l, b: (l, b, 0)),
        out_shape=jax.ShapeDtypeStruct((NLEV, HS // LN, LN), jnp.int32),
    )(tabX)


BP = 128
RW = 16


def _sc_gather_body(BPW, table_ref, idx_ref, off_ref, bary_ref, out_ref,
                    idx_v, off_v, bary_v, rows_v, feats_v, sem):
    NIT = BPW * NLEV
    nc = 2
    wid = lax.axis_index("s") * nc + lax.axis_index("c")
    lane = lax.iota(jnp.int32, 16)
    base_pat = lane * 32
    dup0 = jnp.zeros((16,), jnp.int32)
    msk_hi = np.int32(np.uint32(0xFFFF0000).view(np.int32))

    def load_slabs(it, p):
        bi = it // NLEV
        l = it - bi * NLEV
        blk = wid * BPW + bi
        pb = blk // SB
        sb = blk - pb * SB
        pltpu.sync_copy(idx_ref.at[l, pb, :, sb], idx_v.at[p])
        pltpu.sync_copy(off_ref.at[l, pb, :, sb], off_v.at[p])
        pltpu.sync_copy(bary_ref.at[l, pb, :, sb], bary_v.at[p])

    def fire(p):
        for r in range(DD):
            pltpu.async_copy(table_ref.at[idx_v.at[p, r]],
                             rows_v.at[p, r], sem)

    def drain(p):
        for r in range(DD):
            pltpu.make_async_copy(table_ref.at[idx_v.at[p, r]],
                                  rows_v.at[p, r], sem).wait()

    def compute(it, p):
        bi = it // NLEV
        l = it - bi * NLEV
        blk = wid * BPW + bi
        col = 2 * l
        for g in range(BP // 16):
            boff = 16 * g
            bvec = boff + lane
            widx = base_pat + (g * 512 + col)
            acc0 = jnp.zeros((16,), jnp.float32)
            acc1 = jnp.zeros((16,), jnp.float32)
            for r in range(DD):
                rvec = dup0 + r
                w = bary_v[p, r, pl.ds(boff, 16)]
                off = off_v[p, r, pl.ds(boff, 16)]
                pair = plsc.load_gather(rows_v, [dup0 + p, rvec, bvec, off])
                f0 = plsc.bitcast(lax.shift_left(pair, 16), jnp.float32)
                f1 = plsc.bitcast(pair & msk_hi, jnp.float32)
                acc0 = acc0 + w * f0
                acc1 = acc1 + w * f1
            plsc.store_scatter(feats_v, [widx], acc0)
            plsc.store_scatter(feats_v, [widx + 1], acc1)

        @pl.when(l == NLEV - 1)
        def _():
            pltpu.sync_copy(feats_v, out_ref.at[blk])

    load_slabs(0, 0)
    fire(0)

    def body(t, _):
        a = 2 * t
        b = a + 1
        load_slabs(b, 1)
        fire(1)
        drain(0)
        compute(a, 0)

        @pl.when(b + 1 < NIT)
        def _():
            load_slabs(b + 1, 0)
            fire(0)

        drain(1)
        compute(b, 1)
        return 0

    lax.fori_loop(0, NIT // 2, body, 0)


def _sc_gather(table, idx, off, bary):
    nblk = idx.shape[1] * SB
    mesh = plsc.VectorSubcoreMesh(core_axis_name="c", subcore_axis_name="s")
    f = pl.kernel(
        functools.partial(_sc_gather_body, nblk // 32),
        out_type=jax.ShapeDtypeStruct((nblk, BP * 32), jnp.float32),
        mesh=mesh,
        compiler_params=pltpu.CompilerParams(needs_layout_passes=False,
                                             use_tc_tiling_on_sc=False),
        scratch_types=[
            pltpu.VMEM((2, DD, LN), jnp.int32),
            pltpu.VMEM((2, DD, LN), jnp.int32),
            pltpu.VMEM((2, DD, LN), jnp.float32),
            pltpu.VMEM((2, DD, LN, RW), jnp.int32),
            pltpu.VMEM((BP * 32,), jnp.float32),
            pltpu.SemaphoreType.DMA,
        ],
    )
    return f(table, idx, off, bary)


def _mlp_body(h_ref, w1_ref, b1_ref, w2_ref, b2_ref, w3_ref, b3_ref, o_ref):
    h = h_ref[...]
    h1 = jnp.maximum(jnp.dot(h, w1_ref[...],
                             preferred_element_type=jnp.float32)
                     + b1_ref[...], 0.0)
    h2 = jnp.maximum(jnp.dot(h1, w2_ref[...],
                             preferred_element_type=jnp.float32)
                     + b2_ref[...], 0.0)
    o_ref[...] = jnp.dot(h2, w3_ref[...],
                         preferred_element_type=jnp.float32) + b3_ref[...]


def _mlp(h, W1, b1, W2, b2, W3, b3):
    M = 8192
    npts = h.shape[0]
    grid = (npts // M,)
    full = lambda shape: pl.BlockSpec(shape, lambda i: tuple(0 for _ in shape))
    return pl.pallas_call(
        _mlp_body,
        grid=grid,
        in_specs=[
            pl.BlockSpec((M, 32), lambda i: (i, 0)),
            full((32, 64)), full((1, 64)),
            full((64, 64)), full((1, 64)),
            full((64, 1)), full((1, 1)),
        ],
        out_specs=pl.BlockSpec((M, 1), lambda i: (i, 0)),
        out_shape=jax.ShapeDtypeStruct((npts, 1), jnp.float32),
    )(h, W1, b1.reshape(1, 64), W2, b2.reshape(1, 64), W3, b3.reshape(1, 1))


def kernel(x, z, tables, W1, b1, W2, b2, W3, b3):
    pos = jnp.concatenate([x / 2.0 + 0.5, z], axis=-1)
    posR = pos.T.reshape(D, PB, SB, LN)
    scales = jnp.asarray(_SCALES).reshape(NLEV, 1)
    tabX = (tables.reshape(NLEV, HS // LN, LN, NFEAT)
            .transpose(0, 1, 3, 2).reshape(NLEV, NFEAT * HS // LN, LN))
    packed = _pack(tabX)
    tabL = packed.reshape(NLEV * HS // RW, RW)
    outs = []
    HPB = PB // 4
    for hi in range(4):
        posH = posR[:, hi * HPB:(hi + 1) * HPB]
        idx, off, bary = _encode(posH, scales, HPB)
        feats = _sc_gather(tabL, idx, off, bary)
        h = feats.reshape(HPB * 2048, NLEV * NFEAT)
        outs.append(_mlp(h, W1, b1, W2, b2, W3, b3))
    return jnp.concatenate(outs, axis=0).squeeze(-1)
